# baseline (speedup 1.0000x reference)
import functools

import jax
import jax.numpy as jnp
from jax import lax
from jax.experimental import pallas as pl
from jax.experimental.pallas import tpu as pltpu
from jax.experimental.pallas import tpu_sc as plsc


def _gather_rows(idx_flat, pos_embed, n_rows, d):
    info = plsc.get_sparse_core_info()
    nw = info.num_cores * info.num_subcores
    rows_per_w = n_rows // nw
    chunk = 16
    nbuf = 4
    skew = 2
    n_chunks = rows_per_w // chunk
    n_groups = n_chunks // nbuf
    mesh = plsc.VectorSubcoreMesh(core_axis_name="c", subcore_axis_name="s")

    @functools.partial(
        pl.kernel,
        mesh=mesh,
        out_type=jax.ShapeDtypeStruct((n_rows, d), jnp.float32),
        scratch_types=[
            pltpu.VMEM((rows_per_w,), jnp.int32),
            pltpu.VMEM((nbuf, chunk, d), jnp.float32),
        ]
        + [pltpu.SemaphoreType.DMA] * (2 * nbuf),
    )
    def k(table_hbm, idx_hbm, out_hbm, idx_all, rows, *sems):
        wid = lax.axis_index("s") * info.num_cores + lax.axis_index("c")
        base = wid * rows_per_w
        gsems = sems[:nbuf]
        ssems = sems[nbuf:]

        pltpu.sync_copy(idx_hbm.at[pl.ds(base, rows_per_w)], idx_all)

        def gather(i, b):
            pltpu.async_copy(
                table_hbm.at[idx_all.at[pl.ds(i * chunk, chunk)]],
                rows.at[b],
                gsems[b],
            )

        def wait_gather(i, b):
            pltpu.make_async_copy(
                table_hbm.at[idx_all.at[pl.ds(i * chunk, chunk)]],
                rows.at[b],
                gsems[b],
            ).wait()

        def scatter(i, b):
            pltpu.async_copy(
                rows.at[b], out_hbm.at[pl.ds(base + i * chunk, chunk)], ssems[b]
            )

        def wait_scatter(i, b):
            pltpu.make_async_copy(
                rows.at[b], out_hbm.at[pl.ds(base + i * chunk, chunk)], ssems[b]
            ).wait()

        def group(g, carry):
            for kk in range(nbuf):
                i = nbuf * g + kk
                b = kk

                @pl.when(g > 0)
                def _free_buf():
                    wait_scatter(i - nbuf, b)

                gather(i, b)

                h = i - skew
                bh = (kk + nbuf - skew) % nbuf
                if kk < skew:
                    @pl.when(g > 0)
                    def _drain():
                        wait_gather(h, bh)
                        scatter(h, bh)

                else:
                    wait_gather(h, bh)
                    scatter(h, bh)
            return carry

        lax.fori_loop(0, n_groups, group, 0)
        for kk in range(skew):
            h = n_chunks - skew + kk
            wait_gather(h, h % nbuf)
            scatter(h, h % nbuf)
        for b in range(nbuf):
            i = n_chunks - nbuf + b
            wait_scatter(i, b)

    return k(pos_embed, idx_flat)


def kernel(x, pos_embed):
    if x.ndim == 1:
        x = x[None, :]
    b, l = x.shape
    v, d = pos_embed.shape
    idx_flat = x.reshape(b * l).astype(jnp.int32)
    out = _gather_rows(idx_flat, pos_embed, b * l, d)
    return out.reshape(b, l, d)


# speedup vs baseline: 1.0087x; 1.0087x over previous
import functools

import jax
import jax.numpy as jnp
from jax import lax
from jax.experimental import pallas as pl
from jax.experimental.pallas import tpu as pltpu
from jax.experimental.pallas import tpu_sc as plsc


def _gather_rows(idx_flat, pos_embed, n_rows, d):
    info = plsc.get_sparse_core_info()
    nw = info.num_cores * info.num_subcores
    rows_per_w = n_rows // nw
    chunk = 8
    nbuf = 8
    skew = 7
    n_chunks = rows_per_w // chunk
    n_groups = n_chunks // nbuf
    mesh = plsc.VectorSubcoreMesh(core_axis_name="c", subcore_axis_name="s")

    @functools.partial(
        pl.kernel,
        mesh=mesh,
        out_type=jax.ShapeDtypeStruct((n_rows, d), jnp.float32),
        scratch_types=[
            pltpu.VMEM((rows_per_w,), jnp.int32),
            pltpu.VMEM((nbuf, chunk, d), jnp.float32),
        ]
        + [pltpu.SemaphoreType.DMA] * (2 * nbuf),
    )
    def k(table_hbm, idx_hbm, out_hbm, idx_all, rows, *sems):
        wid = lax.axis_index("s") * info.num_cores + lax.axis_index("c")
        base = wid * rows_per_w
        gsems = sems[:nbuf]
        ssems = sems[nbuf:]

        pltpu.sync_copy(idx_hbm.at[pl.ds(base, rows_per_w)], idx_all)

        def gather(i, b):
            pltpu.async_copy(
                table_hbm.at[idx_all.at[pl.ds(i * chunk, chunk)]],
                rows.at[b],
                gsems[b],
            )

        def wait_gather(i, b):
            pltpu.make_async_copy(
                table_hbm.at[idx_all.at[pl.ds(i * chunk, chunk)]],
                rows.at[b],
                gsems[b],
            ).wait()

        def scatter(i, b):
            pltpu.async_copy(
                rows.at[b], out_hbm.at[pl.ds(base + i * chunk, chunk)], ssems[b]
            )

        def wait_scatter(i, b):
            pltpu.make_async_copy(
                rows.at[b], out_hbm.at[pl.ds(base + i * chunk, chunk)], ssems[b]
            ).wait()

        def group(g, carry):
            for kk in range(nbuf):
                i = nbuf * g + kk
                b = kk

                @pl.when(g > 0)
                def _free_buf():
                    wait_scatter(i - nbuf, b)

                gather(i, b)

                h = i - skew
                bh = (kk + nbuf - skew) % nbuf
                if kk < skew:
                    @pl.when(g > 0)
                    def _drain():
                        wait_gather(h, bh)
                        scatter(h, bh)

                else:
                    wait_gather(h, bh)
                    scatter(h, bh)
            return carry

        lax.fori_loop(0, n_groups, group, 0)
        for kk in range(skew):
            h = n_chunks - skew + kk
            wait_gather(h, h % nbuf)
            scatter(h, h % nbuf)
        for b in range(nbuf):
            i = n_chunks - nbuf + b
            wait_scatter(i, b)

    return k(pos_embed, idx_flat)


def kernel(x, pos_embed):
    if x.ndim == 1:
        x = x[None, :]
    b, l = x.shape
    v, d = pos_embed.shape
    idx_flat = x.reshape(b * l).astype(jnp.int32)
    out = _gather_rows(idx_flat, pos_embed, b * l, d)
    return out.reshape(b, l, d)
